# baseline (speedup 1.0000x reference)
import jax
import jax.numpy as jnp
import numpy as np
from jax import lax
from jax.experimental import pallas as pl
from jax.experimental.pallas import tpu as pltpu
from jax.experimental.pallas import tpu_sc as plsc

_FIELD_DIM = 4000
NF = 26
D = 16
V = NF * _FIELD_DIM
B = 4096
NC, NS = 2, 16
NW = NC * NS
BPW = B // NW
NB = 4
NCHUNK = BPW // NB
NIDX = NB * NF
NROW = NF * NIDX


def _body(idx_hbm, w_hbm, fc_hbm, bias_hbm, out_hbm,
          idxv, ibx0, ibx1, rows0, rows1, fcv0, fcv1, biasv, zbuf,
          sem0, sem1):
    cid = lax.axis_index("c")
    sid = lax.axis_index("s")
    wid = sid * NC + cid
    base_b = wid * BPW

    pltpu.sync_copy(idx_hbm.at[pl.ds(base_b, BPW), :], idxv)
    pltpu.sync_copy(bias_hbm, biasv)

    lane = lax.iota(jnp.int32, 16)
    fc_mask = lane >= 6
    bias_mask = lane == 0

    def build(c, ibx):
        for bb in range(NB):
            b = c * NB + bb
            ibx[pl.ds(bb * NF, 16)] = idxv[b, pl.ds(0, 16)]
            ibx[pl.ds(bb * NF + 10, 16)] = idxv[b, pl.ds(10, 16)]

    def fire(ibx, rows, fcv, sem):
        for t in range(NF):
            pltpu.async_copy(w_hbm.at[pl.ds(t * V, V), :].at[ibx],
                             rows.at[pl.ds(t * NIDX, NIDX), :], sem)
        pltpu.async_copy(fc_hbm.at[ibx], fcv, sem)

    def drain(rows, fcv, sem):
        pltpu.make_async_copy(w_hbm.at[pl.ds(0, NROW), :], rows, sem).wait()
        pltpu.make_async_copy(fc_hbm.at[pl.ds(0, NIDX)], fcv, sem).wait()

    def compute(c, rows, fcv):
        for bb in range(NB):
            boff = bb * NF
            acc = jnp.zeros((16,), jnp.float32)
            for d in range(1, NF):
                @pl.loop(0, NF - d, init_carry=acc, unroll=4)
                def _diag(i, a, _d=d, _boff=boff):
                    return a + (rows[i * (NIDX + 1) + _d * NIDX + _boff]
                                * rows[i * (NIDX + 1) + _d + _boff])
                acc = _diag
            v0 = fcv[pl.ds(boff, 16)]
            v1 = jnp.where(fc_mask, fcv[pl.ds(boff + 10, 16)], 0.0)
            bv = jnp.where(bias_mask, biasv[...], 0.0)
            z = jnp.sum(acc + v0 + v1 + bv)
            zoff = (c // 4) * 16
            pos = (c % 4) * NB + bb
            zvec = zbuf[pl.ds(zoff, 16)]
            zbuf[pl.ds(zoff, 16)] = jnp.where(lane == pos, z, zvec)

    build(0, ibx0)
    fire(ibx0, rows0, fcv0, sem0)

    @pl.loop(0, NCHUNK // 2)
    def _pair(cp):
        c0 = cp * 2
        c1 = c0 + 1
        build(c1, ibx1)
        fire(ibx1, rows1, fcv1, sem1)
        drain(rows0, fcv0, sem0)
        compute(c0, rows0, fcv0)

        @pl.when(c1 + 1 < NCHUNK)
        def _prefetch():
            build(c1 + 1, ibx0)
            fire(ibx0, rows0, fcv0, sem0)

        drain(rows1, fcv1, sem1)
        compute(c1, rows1, fcv1)

    for k in range(BPW // 16):
        zv = zbuf[pl.ds(k * 16, 16)]
        zbuf[pl.ds(k * 16, 16)] = 1.0 / (1.0 + jnp.exp(-zv))
    pltpu.sync_copy(zbuf, out_hbm.at[pl.ds(base_b, BPW)])


@jax.jit
def _ffm(idx, w, fc_flat, bias16):
    mesh = plsc.VectorSubcoreMesh(core_axis_name="c", subcore_axis_name="s",
                                  num_cores=NC, num_subcores=NS)
    return pl.kernel(
        _body,
        out_type=jax.ShapeDtypeStruct((B,), jnp.float32),
        mesh=mesh,
        compiler_params=pltpu.CompilerParams(needs_layout_passes=False,
                                             use_tc_tiling_on_sc=False),
        scratch_types=[
            pltpu.VMEM((BPW, NF), jnp.int32),
            pltpu.VMEM((NIDX,), jnp.int32),
            pltpu.VMEM((NIDX,), jnp.int32),
            pltpu.VMEM((NROW, D), jnp.float32),
            pltpu.VMEM((NROW, D), jnp.float32),
            pltpu.VMEM((NIDX,), jnp.float32),
            pltpu.VMEM((NIDX,), jnp.float32),
            pltpu.VMEM((16,), jnp.float32),
            pltpu.VMEM((BPW,), jnp.float32),
            pltpu.SemaphoreType.DMA,
            pltpu.SemaphoreType.DMA,
        ],
    )(idx, w, fc_flat, bias16)


def kernel(x, fc, bias, W):
    offsets = jnp.asarray(np.arange(NF, dtype=np.int32) * _FIELD_DIM)
    idx = x.astype(jnp.int32) + offsets[None, :]
    fc_flat = fc.reshape(V)
    bias16 = jnp.broadcast_to(bias.astype(jnp.float32), (16,))
    w_a = W.reshape(NF, V // 8, 128)
    w_flat = jax.lax.optimization_barrier(w_a).reshape(NF * V, D)
    return _ffm(idx, w_flat, fc_flat, bias16)


# speedup vs baseline: 1.0692x; 1.0692x over previous
import jax
import jax.numpy as jnp
import numpy as np
from jax import lax
from jax.experimental import pallas as pl
from jax.experimental.pallas import tpu as pltpu
from jax.experimental.pallas import tpu_sc as plsc

_FIELD_DIM = 4000
NF = 26
D = 16
V = NF * _FIELD_DIM
B = 4096
NC, NS = 2, 16
NW = NC * NS
BPW = B // NW
NB = 4
NCHUNK = BPW // NB
NIDX = NB * NF
NROW = NF * NIDX


def _body(idx_hbm, w_hbm, fc_hbm, bias_hbm, out_hbm,
          idxv, ibx0, ibx1, rows0, rows1, fcv0, fcv1, biasv, zbuf,
          sem0, sem1):
    cid = lax.axis_index("c")
    sid = lax.axis_index("s")
    wid = sid * NC + cid
    base_b = wid * BPW

    pltpu.sync_copy(idx_hbm.at[pl.ds(base_b, BPW), :], idxv)
    pltpu.sync_copy(bias_hbm, biasv)

    lane = lax.iota(jnp.int32, 16)
    fc_mask = lane >= 6
    bias_mask = lane == 0

    def build(c, ibx):
        for bb in range(NB):
            b = c * NB + bb
            ibx[pl.ds(bb * NF, 16)] = idxv[b, pl.ds(0, 16)]
            ibx[pl.ds(bb * NF + 10, 16)] = idxv[b, pl.ds(10, 16)]

    def fire(ibx, rows, fcv, sem):
        for t in range(NF):
            pltpu.async_copy(w_hbm.at[pl.ds(t * V, V), :].at[ibx],
                             rows.at[pl.ds(t * NIDX, NIDX), :], sem)
        pltpu.async_copy(fc_hbm.at[ibx], fcv, sem)

    def drain(rows, fcv, sem):
        pltpu.make_async_copy(w_hbm.at[pl.ds(0, NROW), :], rows, sem).wait()
        pltpu.make_async_copy(fc_hbm.at[pl.ds(0, NIDX)], fcv, sem).wait()

    def compute(c, rows, fcv):
        for bb in range(NB):
            boff = bb * NF
            acc = jnp.zeros((16,), jnp.float32)
            for d in range(1, NF):
                @pl.loop(0, NF - d, init_carry=acc, unroll=4)
                def _diag(i, a, _d=d, _boff=boff):
                    return a + (rows[i * (NIDX + 1) + _d * NIDX + _boff]
                                * rows[i * (NIDX + 1) + _d + _boff])
                acc = _diag
            v0 = fcv[pl.ds(boff, 16)]
            v1 = jnp.where(fc_mask, fcv[pl.ds(boff + 10, 16)], 0.0)
            bv = jnp.where(bias_mask, biasv[...], 0.0)
            z = jnp.sum(acc + v0 + v1 + bv)
            zoff = (c // 4) * 16
            pos = (c % 4) * NB + bb
            zvec = zbuf[pl.ds(zoff, 16)]
            zbuf[pl.ds(zoff, 16)] = jnp.where(lane == pos, z, zvec)

    build(0, ibx0)
    fire(ibx0, rows0, fcv0, sem0)

    @pl.loop(0, NCHUNK // 2)
    def _pair(cp):
        c0 = cp * 2
        c1 = c0 + 1
        build(c1, ibx1)
        fire(ibx1, rows1, fcv1, sem1)
        drain(rows0, fcv0, sem0)
        compute(c0, rows0, fcv0)

        @pl.when(c1 + 1 < NCHUNK)
        def _prefetch():
            build(c1 + 1, ibx0)
            fire(ibx0, rows0, fcv0, sem0)

        drain(rows1, fcv1, sem1)
        compute(c1, rows1, fcv1)

    for k in range(BPW // 16):
        zv = zbuf[pl.ds(k * 16, 16)]
        zbuf[pl.ds(k * 16, 16)] = 1.0 / (1.0 + jnp.exp(-zv))
    pltpu.sync_copy(zbuf, out_hbm.at[pl.ds(base_b, BPW)])


NCHK = 102
CW = 1024
CWT = V - (NCHK - 1) * CW
NTASK = NF * NCHK


def _tr_body(wt_hbm, edge_hbm, out_hbm, fbuf0, fbuf1, tbuf, cbuf, ebuf, sem0, sem1):
    cid = lax.axis_index("c")
    sid = lax.axis_index("s")
    wid = sid * NC + cid

    lane = lax.iota(jnp.int32, 16)

    NFULL = NCHK - 1
    NT1 = NF * NFULL
    nround = (NT1 + NW - 1) // NW

    def fire_full(gidx, fbuf, sem):
        t = gidx // NFULL
        c0 = (gidx % NFULL) * CW
        pltpu.async_copy(wt_hbm.at[t, :, pl.ds(c0, CW)], fbuf, sem)

    def drain_full(fbuf, sem):
        pltpu.make_async_copy(wt_hbm.at[0, :, pl.ds(0, CW)], fbuf, sem).wait()

    def emit(gidx, fbuf):
        t = gidx // NFULL
        c0 = (gidx % NFULL) * CW

        @pl.loop(0, CW // 8, unroll=4)
        def _row(r):
            base = r * 8
            for s in range(8):
                vec = plsc.load_gather(fbuf, [lane, jnp.broadcast_to(base + s, (16,))])
                cbuf[r, pl.ds(16 * s, 16)] = vec
        pltpu.sync_copy(cbuf.at[pl.ds(0, CW // 8), :],
                        out_hbm.at[t, pl.ds(pl.multiple_of(c0 // 8, 8), CW // 8), :])

    @pl.when(wid < NT1)
    def _prologue():
        fire_full(wid, fbuf0, sem0)

    @pl.loop(0, nround)
    def _round(g):
        gidx = g * NW + wid
        gidx2 = gidx + NW

        @pl.when((gidx2 < NT1) & (g % 2 == 0))
        def _pf0():
            fire_full(gidx2, fbuf1, sem1)

        @pl.when((gidx2 < NT1) & (g % 2 == 1))
        def _pf1():
            fire_full(gidx2, fbuf0, sem0)

        @pl.when((gidx < NT1) & (g % 2 == 0))
        def _do0():
            drain_full(fbuf0, sem0)
            emit(gidx, fbuf0)

        @pl.when((gidx < NT1) & (g % 2 == 1))
        def _do1():
            drain_full(fbuf1, sem1)
            emit(gidx, fbuf1)

    @pl.when(wid < NF)
    def _tail():
        t = wid
        c0 = NFULL * CW
        pltpu.sync_copy(wt_hbm.at[t, :, pl.ds(c0, 512)], tbuf)

        @pl.loop(0, 512 // 8, unroll=4)
        def _row(r):
            base = r * 8
            for s in range(8):
                vec = plsc.load_gather(tbuf, [lane, jnp.broadcast_to(base + s, (16,))])
                cbuf[r, pl.ds(16 * s, 16)] = vec
        pltpu.sync_copy(cbuf.at[pl.ds(0, 512 // 8), :],
                        out_hbm.at[t, pl.ds(c0 // 8, 512 // 8), :])
        pltpu.sync_copy(edge_hbm.at[t], ebuf)
        pltpu.sync_copy(ebuf, out_hbm.at[t, pl.ds((V - 64) // 8, 8), :])


@jax.jit
def _transpose(w_t, edge_p):
    mesh = plsc.VectorSubcoreMesh(core_axis_name="c", subcore_axis_name="s",
                                  num_cores=NC, num_subcores=NS)
    return pl.kernel(
        _tr_body,
        out_type=jax.ShapeDtypeStruct((NF, V // 8, 128), jnp.float32),
        mesh=mesh,
        compiler_params=pltpu.CompilerParams(needs_layout_passes=False,
                                             use_tc_tiling_on_sc=True),
        scratch_types=[
            pltpu.VMEM((D, CW), jnp.float32),
            pltpu.VMEM((D, CW), jnp.float32),
            pltpu.VMEM((D, 512), jnp.float32),
            pltpu.VMEM((CW // 8, 128), jnp.float32),
            pltpu.VMEM((8, 128), jnp.float32),
            pltpu.SemaphoreType.DMA,
            pltpu.SemaphoreType.DMA,
        ],
    )(w_t, edge_p)


@jax.jit
def _ffm(idx, w, fc_flat, bias16):
    mesh = plsc.VectorSubcoreMesh(core_axis_name="c", subcore_axis_name="s",
                                  num_cores=NC, num_subcores=NS)
    return pl.kernel(
        _body,
        out_type=jax.ShapeDtypeStruct((B,), jnp.float32),
        mesh=mesh,
        compiler_params=pltpu.CompilerParams(needs_layout_passes=False,
                                             use_tc_tiling_on_sc=False),
        scratch_types=[
            pltpu.VMEM((BPW, NF), jnp.int32),
            pltpu.VMEM((NIDX,), jnp.int32),
            pltpu.VMEM((NIDX,), jnp.int32),
            pltpu.VMEM((NROW, D), jnp.float32),
            pltpu.VMEM((NROW, D), jnp.float32),
            pltpu.VMEM((NIDX,), jnp.float32),
            pltpu.VMEM((NIDX,), jnp.float32),
            pltpu.VMEM((16,), jnp.float32),
            pltpu.VMEM((BPW,), jnp.float32),
            pltpu.SemaphoreType.DMA,
            pltpu.SemaphoreType.DMA,
        ],
    )(idx, w, fc_flat, bias16)


def kernel(x, fc, bias, W):
    offsets = jnp.asarray(np.arange(NF, dtype=np.int32) * _FIELD_DIM)
    idx = x.astype(jnp.int32) + offsets[None, :]
    fc_flat = fc.reshape(V)
    bias16 = jnp.broadcast_to(bias.astype(jnp.float32), (16,))
    edge = W[:, V - 64:, :]
    edge_p = edge.reshape(NF, 8, 8, 16).reshape(NF, 8, 128)
    w128 = _transpose(jnp.transpose(W, (0, 2, 1)), edge_p)
    w_flat = w128.reshape(NF * V, D)
    return _ffm(idx, w_flat, fc_flat, bias16)
